# baseline (speedup 1.0000x reference)
import jax
import jax.numpy as jnp
from jax import lax
from jax.experimental import pallas as pl
from jax.experimental.pallas import tpu as pltpu
from jax.experimental.pallas import tpu_sc as plsc

VOCAB = 100000
D_MODEL = 1024
BATCH = 4
SEQ = 2048
N_TOKENS = 20

_NC = 2
_NS = 16
_NW = _NC * _NS
_ROWS = BATCH * SEQ
_RPW = _ROWS // _NW
_CH = 32
_NCHUNK = _RPW // _CH
_WPB = SEQ // _RPW
_PSTRIDE = 32
_PN = 24
_NBUF = 3


def _body(idx_hbm, wte_hbm, learned_hbm, pidx_hbm, out_hbm,
          idx_v, rows0_v, rows1_v, rows2_v, learned_v, pidx_v,
          gsem0, gsem1, gsem2, ssem0, ssem1, ssem2, psem):
    wid = lax.axis_index("s") * _NC + lax.axis_index("c")
    base = wid * _RPW
    owns_prompt = wid % _WPB == 0
    b = wid // _WPB

    pltpu.sync_copy(idx_hbm.at[pl.ds(base, _RPW)], idx_v)

    bufs = (rows0_v, rows1_v, rows2_v)
    gsems = (gsem0, gsem1, gsem2)
    ssems = (ssem0, ssem1, ssem2)

    def gather(j):
        return pltpu.async_copy(
            wte_hbm.at[idx_v.at[pl.ds(j * _CH, _CH)]], bufs[j % _NBUF],
            gsems[j % _NBUF])

    def scatter(j):
        return pltpu.async_copy(
            bufs[j % _NBUF], out_hbm.at[pl.ds(base + j * _CH, _CH)],
            ssems[j % _NBUF])

    g = {j: gather(j) for j in range(_NBUF)}
    s = {}
    prompt = {}
    for j in range(_NCHUNK):
        g[j].wait()
        s[j] = scatter(j)
        if j + _NBUF < _NCHUNK:
            s[j].wait()
            g[j + _NBUF] = gather(j + _NBUF)
        if j == 0:
            if _NBUF >= _NCHUNK:
                s[0].wait()

            @pl.when(owns_prompt)
            def _():
                pltpu.sync_copy(pidx_hbm.at[pl.ds(b * _PSTRIDE, _PN)],
                                pidx_v)
                pltpu.sync_copy(learned_hbm, learned_v)
                prompt[0] = pltpu.async_copy(
                    learned_v, out_hbm.at[pidx_v], psem)
    for j in range(_NCHUNK - _NBUF, _NCHUNK):
        s[j].wait()

    @pl.when(owns_prompt)
    def _():
        prompt[0].wait()


@jax.jit
def _soft_embedding(tokens, wte_weight, learned_embedding):
    idx = tokens.reshape(_ROWS)
    t = jnp.arange(_PSTRIDE, dtype=jnp.int32) % N_TOKENS
    pidx = (jnp.arange(BATCH, dtype=jnp.int32)[:, None] * SEQ + 1 + t[None, :]
            ).reshape(BATCH * _PSTRIDE)
    learned_pad = jnp.concatenate(
        [learned_embedding, learned_embedding[: _PN - N_TOKENS]], axis=0
    )
    mesh = plsc.VectorSubcoreMesh(core_axis_name="c", subcore_axis_name="s")
    out = pl.kernel(
        _body,
        out_type=jax.ShapeDtypeStruct((_ROWS, D_MODEL), jnp.float32),
        mesh=mesh,
        scratch_types=[
            pltpu.VMEM((_RPW,), jnp.int32),
            pltpu.VMEM((_CH, D_MODEL), jnp.float32),
            pltpu.VMEM((_CH, D_MODEL), jnp.float32),
            pltpu.VMEM((_CH, D_MODEL), jnp.float32),
            pltpu.VMEM((_PN, D_MODEL), jnp.float32),
            pltpu.VMEM((_PN,), jnp.int32),
            pltpu.SemaphoreType.DMA,
            pltpu.SemaphoreType.DMA,
            pltpu.SemaphoreType.DMA,
            pltpu.SemaphoreType.DMA,
            pltpu.SemaphoreType.DMA,
            pltpu.SemaphoreType.DMA,
            pltpu.SemaphoreType.DMA,
        ],
    )(idx, wte_weight, learned_pad, pidx)
    return out.reshape(BATCH, SEQ, D_MODEL)


def kernel(tokens, wte_weight, learned_embedding):
    return _soft_embedding(tokens, wte_weight, learned_embedding)
